# baseline (speedup 1.0000x reference)
import functools

import jax
import jax.numpy as jnp
from jax import lax
from jax.experimental import pallas as pl
from jax.experimental.pallas import tpu as pltpu
from jax.experimental.pallas import tpu_sc as plsc

H = 8
D = 2048
DK = D // H
N_CLS = 12
T = 128
B = 8
S = 256
NW = 32


def _mlp_body(x_ref, w1_ref, b1_ref, w2_ref, b2_ref, w3_ref, b3_ref,
              lab_ref, le_ref, re_ref, rl_ref,
              loss_ref, eidx_ref, ridx_ref):
    h1 = jnp.maximum(jnp.dot(x_ref[...], w1_ref[...],
                             preferred_element_type=jnp.float32) + b1_ref[...], 0.0)
    h2 = jnp.maximum(jnp.dot(h1, w2_ref[...],
                             preferred_element_type=jnp.float32) + b2_ref[...], 0.0)
    z = jnp.dot(h2, w3_ref[...], preferred_element_type=jnp.float32) + b3_ref[...]
    s = jax.nn.sigmoid(z)
    m = jnp.max(s, axis=1, keepdims=True)
    e = jnp.exp(s - m)
    logp = s - m - jnp.log(jnp.sum(e, axis=1, keepdims=True))
    cols = lax.broadcasted_iota(jnp.int32, (B, N_CLS), 1)
    labmask = cols == lab_ref[...]
    loss_ref[...] = jnp.sum(jnp.where(labmask, logp, 0.0),
                            keepdims=True).reshape(1, 1) * (-1.0 / B)
    cand = jnp.where(s == m, cols, N_CLS)
    clsv = jnp.min(cand, axis=1, keepdims=True)
    acc_le = jnp.zeros((B, T), jnp.int32)
    acc_re = jnp.zeros((B, T), jnp.int32)
    acc_rl = jnp.zeros((B, T), jnp.int32)
    for c in range(N_CLS):
        msk = clsv == c
        acc_le = jnp.where(msk, le_ref[c:c + 1, :], acc_le)
        acc_re = jnp.where(msk, re_ref[c:c + 1, :], acc_re)
        acc_rl = jnp.where(msk, rl_ref[c:c + 1, :], acc_rl)
    eidx_ref[:, 0:T] = acc_le
    eidx_ref[:, T:2 * T] = acc_re
    ridx_ref[...] = acc_rl


def _mlp_call(x, W1, b1, W2, b2, W3, b3, lab, cls_le, cls_re, cls_rela):
    return pl.pallas_call(
        _mlp_body,
        out_shape=(
            jax.ShapeDtypeStruct((1, 1), jnp.float32),
            jax.ShapeDtypeStruct((B, 2 * T), jnp.int32),
            jax.ShapeDtypeStruct((B, T), jnp.int32),
        ),
    )(x, W1, b1, W2, b2, W3, b3, lab, cls_le, cls_re, cls_rela)


_CH = 16
_NCHUNK = 6


def _sc_gather_body(eidx_hbm, ridx_hbm, etab_hbm, rtab_hbm,
                    e_out, r_out,
                    idx_e, idx_r, b0, b1, b2, g0, g1, g2, w0, w1, w2):
    wid = lax.axis_index("s") * 2 + lax.axis_index("c")
    bufs = (b0, b1, b2)
    gsems = (g0, g1, g2)
    wsems = (w0, w1, w2)
    pltpu.sync_copy(eidx_hbm.at[pl.ds(wid * 64, 64)], idx_e)
    pltpu.sync_copy(ridx_hbm.at[pl.ds(wid * 32, 32)], idx_r)

    def src(i):
        if i < 4:
            return etab_hbm.at[idx_e.at[pl.ds(i * _CH, _CH)]]
        return rtab_hbm.at[idx_r.at[pl.ds((i - 4) * _CH, _CH)]]

    def dst(i):
        if i < 4:
            return e_out.at[pl.ds(wid * 64 + i * _CH, _CH)]
        return r_out.at[pl.ds(wid * 32 + (i - 4) * _CH, _CH)]

    for i in range(3):
        pltpu.async_copy(src(i), bufs[i], gsems[i])
    for i in range(3):
        pltpu.make_async_copy(src(i), bufs[i], gsems[i]).wait()
        pltpu.async_copy(bufs[i], dst(i), wsems[i])
    for i in range(3, _NCHUNK):
        j = i % 3
        pltpu.make_async_copy(bufs[j], dst(i - 3), wsems[j]).wait()
        pltpu.async_copy(src(i), bufs[j], gsems[j])
    for i in range(3, _NCHUNK):
        j = i % 3
        pltpu.make_async_copy(src(i), bufs[j], gsems[j]).wait()
        pltpu.async_copy(bufs[j], dst(i), wsems[j])
    for i in range(3, _NCHUNK):
        j = i % 3
        pltpu.make_async_copy(bufs[j], dst(i), wsems[j]).wait()


def _sc_gather(eidx, ridx, etab, rtab):
    f = pl.kernel(
        _sc_gather_body,
        out_type=(
            jax.ShapeDtypeStruct((B * 2 * T, D), jnp.float32),
            jax.ShapeDtypeStruct((B * T, D), jnp.float32),
        ),
        mesh=plsc.VectorSubcoreMesh(core_axis_name="c", subcore_axis_name="s"),
        scratch_types=[
            pltpu.VMEM((64,), jnp.int32),
            pltpu.VMEM((32,), jnp.int32),
            pltpu.VMEM((_CH, D), jnp.float32),
            pltpu.VMEM((_CH, D), jnp.float32),
            pltpu.VMEM((_CH, D), jnp.float32),
            pltpu.SemaphoreType.DMA,
            pltpu.SemaphoreType.DMA,
            pltpu.SemaphoreType.DMA,
            pltpu.SemaphoreType.DMA,
            pltpu.SemaphoreType.DMA,
            pltpu.SemaphoreType.DMA,
        ],
    )
    return f(eidx, ridx, etab, rtab)


def _cast_body(wq_ref, wk_ref, wv_ref, wo_ref, wg_ref,
               oq_ref, ok_ref, ov_ref, oo_ref, og_ref):
    oq_ref[...] = wq_ref[...].astype(jnp.bfloat16)
    ok_ref[...] = wk_ref[...].astype(jnp.bfloat16)
    ov_ref[...] = wv_ref[...].astype(jnp.bfloat16)
    oo_ref[...] = wo_ref[...].astype(jnp.bfloat16)
    og_ref[...] = wg_ref[...].astype(jnp.bfloat16)


def _cast_call(Wq, Wk, Wv, Wo, Wg):
    return pl.pallas_call(
        _cast_body,
        grid=(8,),
        in_specs=[pl.BlockSpec((DK, D), lambda i: (i, 0))] * 5,
        out_specs=[pl.BlockSpec((DK, D), lambda i: (i, 0))] * 5,
        out_shape=tuple(
            jax.ShapeDtypeStruct((D, D), jnp.bfloat16) for _ in range(5)),
    )(Wq, Wk, Wv, Wo, Wg)


def _mega_body(e_ref, r_ref, wq_ref, bq_ref, wk_ref, bk_ref, wv_ref, bv_ref,
               wo_ref, bo_ref, wg_ref, bg_ref, img_ref,
               out_ref, meano_s, meane_s, pooled_s):
    step = pl.program_id(0)

    @pl.when(step < B)
    def _attn_phase():
        Eb = e_ref[...]
        Rb = r_ref[...]
        Ebb = Eb.astype(jnp.bfloat16)
        Rbb = Rb.astype(jnp.bfloat16)
        Q = jnp.dot(Ebb, wq_ref[...], preferred_element_type=jnp.float32) + bq_ref[...]
        K = jnp.dot(Rbb, wk_ref[...], preferred_element_type=jnp.float32) + bk_ref[...]
        Qb = Q.astype(jnp.bfloat16)
        Kb = K.astype(jnp.bfloat16)
        ws = []
        for h in range(H):
            sl = slice(h * DK, (h + 1) * DK)
            Sc = lax.dot_general(Qb[:, sl], Kb[:, sl], (((1,), (1,)), ((), ())),
                                 preferred_element_type=jnp.float32) * (1.0 / 16.0)
            P = jnp.exp(Sc)
            A = P / jnp.sum(P, axis=1, keepdims=True)
            ws.append(jnp.sum(A, axis=0, keepdims=True) * (1.0 / (2 * T)))
        W = jnp.concatenate(ws, axis=0)
        U = jnp.dot(W.astype(jnp.bfloat16), Rbb,
                    preferred_element_type=jnp.float32)
        P8 = jnp.dot(U.astype(jnp.bfloat16), wv_ref[...],
                     preferred_element_type=jnp.float32)
        rmo = jnp.concatenate(
            [P8[h:h + 1, h * DK:(h + 1) * DK] + bv_ref[:, h * DK:(h + 1) * DK]
             for h in range(H)], axis=0)
        me = jnp.sum(Eb, axis=0, keepdims=True) * (1.0 / (2 * T))
        rme = jnp.concatenate(
            [me[:, j * DK:(j + 1) * DK] for j in range(H)], axis=0)
        bmask = lax.broadcasted_iota(jnp.int32, (H, B, DK), 1) == step
        meano_s[...] = jnp.where(bmask, rmo[:, None, :], meano_s[...])
        meane_s[...] = jnp.where(bmask, rme[:, None, :], meane_s[...])

    @pl.when((step >= B) & (step < 2 * B))
    def _pool_phase():
        j = step - B
        acc = jnp.zeros((B, DK), jnp.float32) + bo_ref[...]
        for jp in range(H):
            acc += jnp.dot(meano_s[jp].astype(jnp.bfloat16),
                           wo_ref[jp * DK:(jp + 1) * DK, :],
                           preferred_element_type=jnp.float32)
        jmask = lax.broadcasted_iota(jnp.int32, (H, B, DK), 0) == j
        me_j = jnp.sum(jnp.where(jmask, meane_s[...], 0.0), axis=0)
        pooled_s[...] = jnp.where(jmask, (acc + me_j)[None], pooled_s[...])

    @pl.when(step >= 2 * B)
    def _gate_phase():
        acc = jnp.zeros((B, DK), jnp.float32) + bg_ref[...]
        for jp in range(H):
            acc += jnp.dot(pooled_s[jp].astype(jnp.bfloat16),
                           wg_ref[jp * DK:(jp + 1) * DK, :],
                           preferred_element_type=jnp.float32)
        g = jax.nn.sigmoid(acc)
        out_ref[...] = img_ref[...] * (1.0 + g[:, None, :])


def _mega_call(E, R, Wqb, bq, Wkb, bk, Wvb, bv, Wob, bo, Wgb, bg, img):
    def _clip(i, lo):
        return jnp.clip(i - lo, 0, H - 1)
    return pl.pallas_call(
        _mega_body,
        grid=(3 * B,),
        in_specs=[
            pl.BlockSpec((2 * T, D), lambda i: (jnp.clip(i, 0, B - 1), 0)),
            pl.BlockSpec((T, D), lambda i: (jnp.clip(i, 0, B - 1), 0)),
            pl.BlockSpec((D, D), lambda i: (0, 0)),
            pl.BlockSpec((1, D), lambda i: (0, 0)),
            pl.BlockSpec((D, D), lambda i: (0, 0)),
            pl.BlockSpec((1, D), lambda i: (0, 0)),
            pl.BlockSpec((D, D), lambda i: (0, 0)),
            pl.BlockSpec((1, D), lambda i: (0, 0)),
            pl.BlockSpec((D, DK), lambda i: (0, _clip(i, B))),
            pl.BlockSpec((1, DK), lambda i: (0, _clip(i, B))),
            pl.BlockSpec((D, DK), lambda i: (0, _clip(i, 2 * B))),
            pl.BlockSpec((1, DK), lambda i: (0, _clip(i, 2 * B))),
            pl.BlockSpec((B, S, DK), lambda i: (0, 0, _clip(i, 2 * B))),
        ],
        out_specs=pl.BlockSpec((B, S, DK), lambda i: (0, 0, _clip(i, 2 * B))),
        out_shape=jax.ShapeDtypeStruct((B, S, D), jnp.float32),
        scratch_shapes=[
            pltpu.VMEM((H, B, DK), jnp.float32),
            pltpu.VMEM((H, B, DK), jnp.float32),
            pltpu.VMEM((H, B, DK), jnp.float32),
        ],
    )(E, R, Wqb, bq, Wkb, bk, Wvb, bv, Wob, bo, Wgb, bg, img)


def kernel(x, imageFeature, clsLabel, entitysEmbed, relaEmbed,
           cls_le, cls_re, cls_rela,
           W1, b1, W2, b2, W3, b3, Wq, bq, Wk, bk, Wv, bv, Wo, bo, Wg, bg):
    lab = clsLabel.astype(jnp.int32).reshape(B, 1)
    loss, eidx, ridx = _mlp_call(
        x, W1, b1.reshape(1, -1), W2, b2.reshape(1, -1), W3, b3.reshape(1, -1),
        lab, cls_le.astype(jnp.int32), cls_re.astype(jnp.int32),
        cls_rela.astype(jnp.int32))
    Wqb, Wkb, Wvb, Wob, Wgb = _cast_call(Wq, Wk, Wv, Wo, Wg)
    E, R = _sc_gather(eidx.reshape(-1), ridx.reshape(-1),
                      entitysEmbed, relaEmbed)
    out = _mega_call(E, R, Wqb, bq.reshape(1, -1), Wkb, bk.reshape(1, -1),
                     Wvb, bv.reshape(1, -1), Wob, bo.reshape(1, -1),
                     Wgb, bg.reshape(1, -1), imageFeature)
    return out, loss.reshape(())


# speedup vs baseline: 1.1096x; 1.1096x over previous
import jax
import jax.numpy as jnp
from jax import lax
from jax.experimental import pallas as pl
from jax.experimental.pallas import tpu as pltpu
from jax.experimental.pallas import tpu_sc as plsc

H = 8
D = 2048
DK = D // H
N_CLS = 12
T = 128
B = 8
S = 256
NW = 32


def _mlp_body(x_ref, w1_ref, b1_ref, w2_ref, b2_ref, w3_ref, b3_ref,
              lab_ref, le_ref, re_ref, rl_ref,
              loss_ref, eidx_ref, ridx_ref):
    h1 = jnp.maximum(jnp.dot(x_ref[...], w1_ref[...],
                             preferred_element_type=jnp.float32) + b1_ref[...], 0.0)
    h2 = jnp.maximum(jnp.dot(h1, w2_ref[...],
                             preferred_element_type=jnp.float32) + b2_ref[...], 0.0)
    z = jnp.dot(h2, w3_ref[...], preferred_element_type=jnp.float32) + b3_ref[...]
    s = jax.nn.sigmoid(z)
    m = jnp.max(s, axis=1, keepdims=True)
    e = jnp.exp(s - m)
    logp = s - m - jnp.log(jnp.sum(e, axis=1, keepdims=True))
    cols = lax.broadcasted_iota(jnp.int32, (B, N_CLS), 1)
    labmask = cols == lab_ref[...]
    loss_ref[...] = jnp.sum(jnp.where(labmask, logp, 0.0),
                            keepdims=True).reshape(1, 1) * (-1.0 / B)
    cand = jnp.where(s == m, cols, N_CLS)
    clsv = jnp.min(cand, axis=1, keepdims=True)
    acc_le = jnp.zeros((B, T), jnp.int32)
    acc_re = jnp.zeros((B, T), jnp.int32)
    acc_rl = jnp.zeros((B, T), jnp.int32)
    for c in range(N_CLS):
        msk = clsv == c
        acc_le = jnp.where(msk, le_ref[c:c + 1, :], acc_le)
        acc_re = jnp.where(msk, re_ref[c:c + 1, :], acc_re)
        acc_rl = jnp.where(msk, rl_ref[c:c + 1, :], acc_rl)
    for b in range(B):
        eidx_ref[:, b * 2 * T:b * 2 * T + T] = acc_le[b:b + 1, :]
        eidx_ref[:, b * 2 * T + T:(b + 1) * 2 * T] = acc_re[b:b + 1, :]
        ridx_ref[:, b * T:(b + 1) * T] = acc_rl[b:b + 1, :]


def _mlp_call(x, W1, b1, W2, b2, W3, b3, lab, cls_le, cls_re, cls_rela):
    return pl.pallas_call(
        _mlp_body,
        out_shape=(
            jax.ShapeDtypeStruct((1, 1), jnp.float32),
            jax.ShapeDtypeStruct((1, B * 2 * T), jnp.int32),
            jax.ShapeDtypeStruct((1, B * T), jnp.int32),
        ),
    )(x, W1, b1, W2, b2, W3, b3, lab, cls_le, cls_re, cls_rela)


def _sc_gather_body(eidx_hbm, ridx_hbm, etab_hbm, rtab_hbm,
                    e_out, r_out, idx_v, rows_v, sem):
    wid = lax.axis_index("s") * 2 + lax.axis_index("c")
    for chunk in range(2):
        base = wid * 64 + chunk * 32
        pltpu.sync_copy(eidx_hbm.at[0, pl.ds(base, 32)], idx_v)
        pltpu.async_copy(etab_hbm.at[idx_v], rows_v, sem).wait()
        pltpu.sync_copy(rows_v, e_out.at[pl.ds(base, 32)])
    base = wid * 32
    pltpu.sync_copy(ridx_hbm.at[0, pl.ds(base, 32)], idx_v)
    pltpu.async_copy(rtab_hbm.at[idx_v], rows_v, sem).wait()
    pltpu.sync_copy(rows_v, r_out.at[pl.ds(base, 32)])


def _sc_gather(eidx, ridx, etab, rtab):
    f = pl.kernel(
        _sc_gather_body,
        out_type=(
            jax.ShapeDtypeStruct((B * 2 * T, D), jnp.float32),
            jax.ShapeDtypeStruct((B * T, D), jnp.float32),
        ),
        mesh=plsc.VectorSubcoreMesh(core_axis_name="c", subcore_axis_name="s"),
        scratch_types=[
            pltpu.VMEM((32,), jnp.int32),
            pltpu.VMEM((32, D), jnp.float32),
            pltpu.SemaphoreType.DMA,
        ],
    )
    return f(eidx, ridx, etab, rtab)


def _cast_body(wq_ref, wk_ref, oq_ref, ok_ref):
    oq_ref[...] = wq_ref[...].astype(jnp.bfloat16)
    ok_ref[...] = wk_ref[...].astype(jnp.bfloat16)


def _cast_call(Wq, Wk):
    return pl.pallas_call(
        _cast_body,
        grid=(8,),
        in_specs=[pl.BlockSpec((DK, D), lambda i: (i, 0))] * 2,
        out_specs=[pl.BlockSpec((DK, D), lambda i: (i, 0))] * 2,
        out_shape=(
            jax.ShapeDtypeStruct((D, D), jnp.bfloat16),
            jax.ShapeDtypeStruct((D, D), jnp.bfloat16),
        ),
    )(Wq, Wk)


def _attn_body(e_ref, r_ref, wq_ref, bq_ref, wk_ref, bk_ref, wv_ref, bv_ref,
               meano_ref, meane_ref):
    Eb = e_ref[...]
    Rb = r_ref[...]
    Ebb = Eb.astype(jnp.bfloat16)
    Rbb = Rb.astype(jnp.bfloat16)
    Q = jnp.dot(Ebb, wq_ref[...], preferred_element_type=jnp.float32) + bq_ref[...]
    K = jnp.dot(Rbb, wk_ref[...], preferred_element_type=jnp.float32) + bk_ref[...]
    Qb = Q.astype(jnp.bfloat16)
    Kb = K.astype(jnp.bfloat16)
    ws = []
    for h in range(H):
        sl = slice(h * DK, (h + 1) * DK)
        Sc = lax.dot_general(Qb[:, sl], Kb[:, sl], (((1,), (1,)), ((), ())),
                             preferred_element_type=jnp.float32) * (1.0 / 16.0)
        P = jnp.exp(Sc)
        A = P / jnp.sum(P, axis=1, keepdims=True)
        ws.append(jnp.sum(A, axis=0, keepdims=True) * (1.0 / (2 * T)))
    W = jnp.concatenate(ws, axis=0)
    U = jnp.dot(W, Rb, preferred_element_type=jnp.float32)
    P8 = jnp.dot(U, wv_ref[...], preferred_element_type=jnp.float32)
    hsel = (lax.broadcasted_iota(jnp.int32, (H, D), 1) // DK ==
            lax.broadcasted_iota(jnp.int32, (H, D), 0))
    mo = jnp.sum(jnp.where(hsel, P8, 0.0), axis=0, keepdims=True)
    meano_ref[...] = (mo + bv_ref[...])[None]
    meane_ref[...] = (jnp.sum(Eb, axis=0, keepdims=True) * (1.0 / (2 * T)))[None]


def _attn_call(E, R, Wqb, bq, Wkb, bk, Wv, bv):
    return pl.pallas_call(
        _attn_body,
        grid=(B,),
        in_specs=[
            pl.BlockSpec((2 * T, D), lambda b: (b, 0)),
            pl.BlockSpec((T, D), lambda b: (b, 0)),
            pl.BlockSpec((D, D), lambda b: (0, 0)),
            pl.BlockSpec((1, D), lambda b: (0, 0)),
            pl.BlockSpec((D, D), lambda b: (0, 0)),
            pl.BlockSpec((1, D), lambda b: (0, 0)),
            pl.BlockSpec((D, D), lambda b: (0, 0)),
            pl.BlockSpec((1, D), lambda b: (0, 0)),
        ],
        out_specs=[
            pl.BlockSpec((1, 1, D), lambda b: (b, 0, 0)),
            pl.BlockSpec((1, 1, D), lambda b: (b, 0, 0)),
        ],
        out_shape=(
            jax.ShapeDtypeStruct((B, 1, D), jnp.float32),
            jax.ShapeDtypeStruct((B, 1, D), jnp.float32),
        ),
    )(E, R, Wqb, bq, Wkb, bk, Wv, bv)


def _tail_body(meano_ref, meane_ref, wo_ref, bo_ref, wg_ref, bg_ref, img_ref,
               out_ref, pooled_s):
    step = pl.program_id(0)

    @pl.when(step < H)
    def _pool_phase():
        mo = jnp.reshape(meano_ref[...], (B, D))
        acc = jnp.zeros((B, DK), jnp.float32) + bo_ref[...]
        wob = wo_ref[...].astype(jnp.bfloat16)
        for jp in range(H):
            acc += jnp.dot(mo[:, jp * DK:(jp + 1) * DK].astype(jnp.bfloat16),
                           wob[jp * DK:(jp + 1) * DK, :],
                           preferred_element_type=jnp.float32)
        jmask = lax.broadcasted_iota(jnp.int32, (H, B, DK), 0) == step
        pooled_s[...] = jnp.where(jmask, acc[None], pooled_s[...])

    @pl.when(step >= H)
    def _gate_phase():
        me = jnp.reshape(meane_ref[...], (B, D)).astype(jnp.bfloat16)
        wgb = wg_ref[...].astype(jnp.bfloat16)
        acc = jnp.zeros((B, DK), jnp.float32) + bg_ref[...]
        acc += jnp.dot(me, wgb, preferred_element_type=jnp.float32)
        for jp in range(H):
            acc += jnp.dot(pooled_s[jp].astype(jnp.bfloat16),
                           wgb[jp * DK:(jp + 1) * DK, :],
                           preferred_element_type=jnp.float32)
        g = jax.nn.sigmoid(acc)
        out_ref[...] = img_ref[...] * (1.0 + g[:, None, :])


def _tail_call(meanO, meanE, Wo, bo, Wg, bg, img):
    c0 = lambda i: jnp.clip(i, 0, H - 1)
    c1 = lambda i: jnp.clip(i - H, 0, H - 1)
    return pl.pallas_call(
        _tail_body,
        grid=(2 * H,),
        in_specs=[
            pl.BlockSpec((B, 1, D), lambda i: (0, 0, 0)),
            pl.BlockSpec((B, 1, D), lambda i: (0, 0, 0)),
            pl.BlockSpec((D, DK), lambda i: (0, c0(i))),
            pl.BlockSpec((1, DK), lambda i: (0, c0(i))),
            pl.BlockSpec((D, DK), lambda i: (0, c1(i))),
            pl.BlockSpec((1, DK), lambda i: (0, c1(i))),
            pl.BlockSpec((B, S, DK), lambda i: (0, 0, c1(i))),
        ],
        out_specs=pl.BlockSpec((B, S, DK), lambda i: (0, 0, c1(i))),
        out_shape=jax.ShapeDtypeStruct((B, S, D), jnp.float32),
        scratch_shapes=[pltpu.VMEM((H, B, DK), jnp.float32)],
    )(meanO, meanE, Wo, bo, Wg, bg, img)


def kernel(x, imageFeature, clsLabel, entitysEmbed, relaEmbed,
           cls_le, cls_re, cls_rela,
           W1, b1, W2, b2, W3, b3, Wq, bq, Wk, bk, Wv, bv, Wo, bo, Wg, bg):
    lab = clsLabel.astype(jnp.int32).reshape(B, 1)
    loss, eidx, ridx = _mlp_call(
        x, W1, b1.reshape(1, -1), W2, b2.reshape(1, -1), W3, b3.reshape(1, -1),
        lab, cls_le.astype(jnp.int32), cls_re.astype(jnp.int32),
        cls_rela.astype(jnp.int32))
    Wqb, Wkb = _cast_call(Wq, Wk)
    E, R = _sc_gather(eidx, ridx, entitysEmbed, relaEmbed)
    meanO, meanE = _attn_call(E, R, Wqb, bq.reshape(1, -1), Wkb,
                              bk.reshape(1, -1), Wv, bv.reshape(1, -1))
    out = _tail_call(meanO, meanE, Wo, bo.reshape(1, -1),
                     Wg, bg.reshape(1, -1), imageFeature)
    return out, loss.reshape(())
